# baseline (speedup 1.0000x reference)
import jax
import jax.numpy as jnp
from jax import lax
from jax.experimental import pallas as pl

VOCAB_SIZE = 1000


def _body(xt_ref, out_ref):
    idx = xt_ref[...]
    kio = lax.broadcasted_iota(jnp.int32, (1, 200, 4096), 1) + pl.program_id(1) * 200
    out_ref[...] = (kio == idx).astype(jnp.float32)


_one_hot_t = pl.pallas_call(
    _body,
    out_shape=jax.ShapeDtypeStruct((26, VOCAB_SIZE, 4096), jnp.float32),
    grid=(26, 5),
    in_specs=[pl.BlockSpec((1, 1, 4096), lambda j, k: (j, 0, 0))],
    out_specs=pl.BlockSpec((1, 200, 4096), lambda j, k: (j, k, 0)),
)


def kernel(x):
    xt = x.astype(jnp.int32).T.reshape(26, 1, 4096)
    y = _one_hot_t(xt)
    return jnp.transpose(y, (2, 0, 1))
